# baseline (speedup 1.0000x reference)
import jax
import jax.numpy as jnp
from jax import lax
from jax.experimental import pallas as pl
from jax.experimental.pallas import tpu as pltpu, tpu_sc as plsc

_M, _N = 16384, 1024
_NC, _NS, _L = 2, 16, 16
_NW = _NC * _NS
_ROWS_W = _M // _NW
_CR = 32
_NBUF = 2
_LOOK = 1
_NCHUNKS = _ROWS_W // _CR
_VPR = _N // _L


def _sc_body(x_hbm, o_hbm, *scratch):
    bufs = scratch[:_NBUF]
    isems = scratch[_NBUF:2 * _NBUF]
    osems = scratch[2 * _NBUF:3 * _NBUF]
    wid = lax.axis_index("s") * _NC + lax.axis_index("c")
    base = wid * _ROWS_W

    def in_slice(c):
        return x_hbm.at[pl.ds(base + c * _CR, _CR), :]

    def out_slice(c):
        return o_hbm.at[pl.ds(base + c * _CR, _CR), :]

    for k in range(_LOOK):
        pltpu.async_copy(in_slice(k), bufs[k % _NBUF], isems[k % _NBUF])

    @pl.loop(0, _NCHUNKS, step=_NBUF)
    def _ring(c0):
        for b in range(_NBUF):
            c = c0 + b
            pltpu.make_async_copy(in_slice(c), bufs[b], isems[b]).wait()

            @pl.loop(0, _CR)
            def _rows(r, buf=bufs[b]):
                @plsc.parallel_loop(0, _VPR, unroll=8)
                def _vecs(j):
                    v = buf[r, pl.ds(j * _L, _L)]
                    buf[r, pl.ds(j * _L, _L)] = v + v - 1.0

            pltpu.async_copy(bufs[b], out_slice(c), osems[b])
            k = c + _LOOK
            kb = (b + _LOOK) % _NBUF

            @pl.when(k < _NCHUNKS)
            def _prefetch():
                @pl.when(k >= _NBUF)
                def _reclaim():
                    pltpu.make_async_copy(
                        bufs[kb], out_slice(k - _NBUF), osems[kb]
                    ).wait()

                pltpu.async_copy(in_slice(k), bufs[kb], isems[kb])

    for c in range(_NCHUNKS - _NBUF, _NCHUNKS):
        b = c % _NBUF
        pltpu.make_async_copy(bufs[b], out_slice(c), osems[b]).wait()


@jax.jit
def kernel(x):
    mesh = plsc.VectorSubcoreMesh(core_axis_name="c", subcore_axis_name="s")
    return pl.kernel(
        _sc_body,
        out_type=jax.ShapeDtypeStruct((_M, _N), jnp.float32),
        mesh=mesh,
        scratch_types=(
            [pltpu.VMEM((_CR, _N), jnp.float32) for _ in range(_NBUF)]
            + [pltpu.SemaphoreType.DMA for _ in range(2 * _NBUF)]
        ),
    )(x)


# speedup vs baseline: 1.3599x; 1.3599x over previous
import jax
import jax.numpy as jnp
from jax import lax
from jax.experimental import pallas as pl
from jax.experimental.pallas import tpu as pltpu, tpu_sc as plsc

_M, _N = 16384, 1024
_NC, _NS, _L = 2, 16, 16
_NW = _NC * _NS
_ROWS_W = _M // _NW
_CR = 16
_NBUF = 4
_LOOK = 2
_NCHUNKS = _ROWS_W // _CR
_VPR = _N // _L


def _sc_body(x_hbm, o_hbm, *scratch):
    bufs = scratch[:_NBUF]
    isems = scratch[_NBUF:2 * _NBUF]
    osems = scratch[2 * _NBUF:3 * _NBUF]
    wid = lax.axis_index("s") * _NC + lax.axis_index("c")
    base = wid * _ROWS_W

    def in_slice(c):
        return x_hbm.at[pl.ds(base + c * _CR, _CR), :]

    def out_slice(c):
        return o_hbm.at[pl.ds(base + c * _CR, _CR), :]

    for k in range(_LOOK):
        pltpu.async_copy(in_slice(k), bufs[k % _NBUF], isems[k % _NBUF])

    @pl.loop(0, _NCHUNKS, step=_NBUF)
    def _ring(c0):
        for b in range(_NBUF):
            c = c0 + b
            pltpu.make_async_copy(in_slice(c), bufs[b], isems[b]).wait()
            k = c + _LOOK
            kb = (b + _LOOK) % _NBUF

            @pl.when(k < _NCHUNKS)
            def _prefetch():
                @pl.when(k >= _NBUF)
                def _reclaim():
                    pltpu.make_async_copy(
                        bufs[kb], out_slice(k - _NBUF), osems[kb]
                    ).wait()

                pltpu.async_copy(in_slice(k), bufs[kb], isems[kb])

            @pl.loop(0, _CR)
            def _rows(r, buf=bufs[b]):
                @plsc.parallel_loop(0, _VPR, unroll=8)
                def _vecs(j):
                    v = buf[r, pl.ds(j * _L, _L)]
                    buf[r, pl.ds(j * _L, _L)] = v + v - 1.0

            pltpu.async_copy(bufs[b], out_slice(c), osems[b])

    for c in range(_NCHUNKS - _NBUF, _NCHUNKS):
        b = c % _NBUF
        pltpu.make_async_copy(bufs[b], out_slice(c), osems[b]).wait()


@jax.jit
def kernel(x):
    mesh = plsc.VectorSubcoreMesh(core_axis_name="c", subcore_axis_name="s")
    return pl.kernel(
        _sc_body,
        out_type=jax.ShapeDtypeStruct((_M, _N), jnp.float32),
        mesh=mesh,
        scratch_types=(
            [pltpu.VMEM((_CR, _N), jnp.float32) for _ in range(_NBUF)]
            + [pltpu.SemaphoreType.DMA for _ in range(2 * _NBUF)]
        ),
    )(x)


# speedup vs baseline: 1.3662x; 1.0047x over previous
import jax
import jax.numpy as jnp
from jax import lax
from jax.experimental import pallas as pl
from jax.experimental.pallas import tpu as pltpu, tpu_sc as plsc

_M, _N = 16384, 1024
_NC, _NS, _L = 2, 16, 16
_NW = _NC * _NS
_ROWS_W = _M // _NW
_CR = 8
_NBUF = 8
_LOOK = 4
_NCHUNKS = _ROWS_W // _CR
_VPR = _N // _L


def _sc_body(x_hbm, o_hbm, *scratch):
    bufs = scratch[:_NBUF]
    isems = scratch[_NBUF:2 * _NBUF]
    osems = scratch[2 * _NBUF:3 * _NBUF]
    wid = lax.axis_index("s") * _NC + lax.axis_index("c")
    base = wid * _ROWS_W

    def in_slice(c):
        return x_hbm.at[pl.ds(base + c * _CR, _CR), :]

    def out_slice(c):
        return o_hbm.at[pl.ds(base + c * _CR, _CR), :]

    for k in range(_LOOK):
        pltpu.async_copy(in_slice(k), bufs[k % _NBUF], isems[k % _NBUF])

    @pl.loop(0, _NCHUNKS, step=_NBUF)
    def _ring(c0):
        for b in range(_NBUF):
            c = c0 + b
            pltpu.make_async_copy(in_slice(c), bufs[b], isems[b]).wait()
            k = c + _LOOK
            kb = (b + _LOOK) % _NBUF

            @pl.when(k < _NCHUNKS)
            def _prefetch():
                @pl.when(k >= _NBUF)
                def _reclaim():
                    pltpu.make_async_copy(
                        bufs[kb], out_slice(k - _NBUF), osems[kb]
                    ).wait()

                pltpu.async_copy(in_slice(k), bufs[kb], isems[kb])

            @pl.loop(0, _CR)
            def _rows(r, buf=bufs[b]):
                @plsc.parallel_loop(0, _VPR, unroll=8)
                def _vecs(j):
                    v = buf[r, pl.ds(j * _L, _L)]
                    buf[r, pl.ds(j * _L, _L)] = v + v - 1.0

            pltpu.async_copy(bufs[b], out_slice(c), osems[b])

    for c in range(_NCHUNKS - _NBUF, _NCHUNKS):
        b = c % _NBUF
        pltpu.make_async_copy(bufs[b], out_slice(c), osems[b]).wait()


@jax.jit
def kernel(x):
    mesh = plsc.VectorSubcoreMesh(core_axis_name="c", subcore_axis_name="s")
    return pl.kernel(
        _sc_body,
        out_type=jax.ShapeDtypeStruct((_M, _N), jnp.float32),
        mesh=mesh,
        scratch_types=(
            [pltpu.VMEM((_CR, _N), jnp.float32) for _ in range(_NBUF)]
            + [pltpu.SemaphoreType.DMA for _ in range(2 * _NBUF)]
        ),
    )(x)


# speedup vs baseline: 1.3856x; 1.0141x over previous
import jax
import jax.numpy as jnp
from jax import lax
from jax.experimental import pallas as pl
from jax.experimental.pallas import tpu as pltpu, tpu_sc as plsc

_M, _N = 16384, 1024
_NC, _NS, _L = 2, 16, 16
_NW = _NC * _NS
_ROWS_W = _M // _NW
_CR = 8
_NBUF = 8
_LOOK = 6
_NCHUNKS = _ROWS_W // _CR
_VPR = _N // _L


def _sc_body(x_hbm, o_hbm, *scratch):
    bufs = scratch[:_NBUF]
    isems = scratch[_NBUF:2 * _NBUF]
    osems = scratch[2 * _NBUF:3 * _NBUF]
    wid = lax.axis_index("s") * _NC + lax.axis_index("c")
    base = wid * _ROWS_W

    def in_slice(c):
        return x_hbm.at[pl.ds(base + c * _CR, _CR), :]

    def out_slice(c):
        return o_hbm.at[pl.ds(base + c * _CR, _CR), :]

    for k in range(_LOOK):
        pltpu.async_copy(in_slice(k), bufs[k % _NBUF], isems[k % _NBUF])

    @pl.loop(0, _NCHUNKS, step=_NBUF)
    def _ring(c0):
        for b in range(_NBUF):
            c = c0 + b
            pltpu.make_async_copy(in_slice(c), bufs[b], isems[b]).wait()
            k = c + _LOOK
            kb = (b + _LOOK) % _NBUF

            @pl.when(k < _NCHUNKS)
            def _prefetch():
                @pl.when(k >= _NBUF)
                def _reclaim():
                    pltpu.make_async_copy(
                        bufs[kb], out_slice(k - _NBUF), osems[kb]
                    ).wait()

                pltpu.async_copy(in_slice(k), bufs[kb], isems[kb])

            @pl.loop(0, _CR)
            def _rows(r, buf=bufs[b]):
                @plsc.parallel_loop(0, _VPR, unroll=8)
                def _vecs(j):
                    v = buf[r, pl.ds(j * _L, _L)]
                    buf[r, pl.ds(j * _L, _L)] = v + v - 1.0

            pltpu.async_copy(bufs[b], out_slice(c), osems[b])

    for c in range(_NCHUNKS - _NBUF, _NCHUNKS):
        b = c % _NBUF
        pltpu.make_async_copy(bufs[b], out_slice(c), osems[b]).wait()


@jax.jit
def kernel(x):
    mesh = plsc.VectorSubcoreMesh(core_axis_name="c", subcore_axis_name="s")
    return pl.kernel(
        _sc_body,
        out_type=jax.ShapeDtypeStruct((_M, _N), jnp.float32),
        mesh=mesh,
        scratch_types=(
            [pltpu.VMEM((_CR, _N), jnp.float32) for _ in range(_NBUF)]
            + [pltpu.SemaphoreType.DMA for _ in range(2 * _NBUF)]
        ),
    )(x)
